# baseline (speedup 1.0000x reference)
import functools

import numpy as np
import jax
import jax.numpy as jnp
from jax import lax
from jax.experimental import pallas as pl
from jax.experimental.pallas import tpu as pltpu
from jax.experimental.pallas import tpu_sc as plsc

NV = 25000
NC_NODES = 25000
D = 128
DW = D // 2
DEG = 16
NW = 32
PAD_N = 25600
ROWS_PER_W = PAD_N // NW
BLK = 8
NBLK = ROWS_PER_W // BLK
ROW_BLOCK = 1000
GRID = NV // ROW_BLOCK

_P = np.zeros(D, np.int64)
for _c in range(D // 32):
    _P[_c * 32:_c * 32 + 16] = _c * 32 + 2 * np.arange(16)
    _P[_c * 32 + 16:_c * 32 + 32] = _c * 32 + 2 * np.arange(16) + 1


RING = 4
N0 = 120
N1 = 80
_HI = np.int32(-65536)


def _gather_pipeline(table_hbm, idx_v, out_hbm, base, nblk_w,
                     bufs, gs, sos, out_v):
    for k in range(RING):
        pltpu.async_copy(table_hbm.at[idx_v.at[k]], bufs[k], gs[k])

    def compute_block(b, k, first_round):
        pltpu.make_async_copy(table_hbm.at[idx_v.at[b]], bufs[k],
                              gs[k]).wait()
        if first_round:
            drain = pl.when(b >= RING)
        else:
            drain = lambda f: f()

        @drain
        def _():
            pltpu.make_async_copy(out_v.at[pl.ds(k * BLK, BLK)],
                                  out_hbm.at[pl.ds(base, BLK)],
                                  sos[k]).wait()

        buf = bufs[k]

        def row_body(r, carry):
            for g in range(D // 16):
                vs = [buf[r * DEG + j, pl.ds(g * 16, 16)]
                      for j in range(DEG)]
                while len(vs) > 1:
                    vs = [vs[i] + vs[i + 1] for i in range(0, len(vs), 2)]
                out_v[k * BLK + r, pl.ds(g * 16, 16)] = vs[0]
            return carry

        lax.fori_loop(0, BLK, row_body, 0)
        pltpu.async_copy(out_v.at[pl.ds(k * BLK, BLK)],
                         out_hbm.at[pl.ds(base + b * BLK, BLK)], sos[k])

    def grp(q, carry):
        for k in range(RING):
            b = q * RING + k
            compute_block(b, k, first_round=True)

            @pl.when(b + RING < nblk_w)
            def _():
                pltpu.async_copy(table_hbm.at[idx_v.at[b + RING]],
                                 bufs[k], gs[k])
        return carry

    lax.fori_loop(0, nblk_w // RING, grp, 0)
    for k in range(RING):
        pltpu.make_async_copy(out_v.at[pl.ds(k * BLK, BLK)],
                              out_hbm.at[pl.ds(base, BLK)], sos[k]).wait()


def _gather_sum_body(table_hbm, idx_hbm, out_hbm, idx_v,
                     b0, b1, b2, b3, out_v,
                     g0, g1, g2, g3, so0, so1, so2, so3):
    c = lax.axis_index("c")
    s_ = lax.axis_index("s")
    start_blk = jnp.where(c == 0, s_ * N0, 16 * N0 + s_ * N1)
    nblk_w = jnp.where(c == 0, N0, N1)
    base = start_blk * BLK
    pltpu.sync_copy(idx_hbm.at[pl.ds(start_blk, N0)], idx_v)
    _gather_pipeline(table_hbm, idx_v, out_hbm, base, nblk_w,
                     [b0, b1, b2, b3], [g0, g1, g2, g3],
                     [so0, so1, so2, so3], out_v)


@functools.partial(jax.jit)
def _gather_sum(table, idx_pages):
    mesh = plsc.VectorSubcoreMesh(core_axis_name="c", subcore_axis_name="s")
    f = pl.kernel(
        _gather_sum_body,
        mesh=mesh,
        out_type=jax.ShapeDtypeStruct((PAD_N, D), jnp.float32),
        scratch_types=[
            pltpu.VMEM((N0, BLK * DEG), jnp.int32),
            pltpu.VMEM((BLK * DEG, D), jnp.float32),
            pltpu.VMEM((BLK * DEG, D), jnp.float32),
            pltpu.VMEM((BLK * DEG, D), jnp.float32),
            pltpu.VMEM((BLK * DEG, D), jnp.float32),
            pltpu.VMEM((RING * BLK, D), jnp.float32),
        ] + [pltpu.SemaphoreType.DMA] * 8,
    )
    return f(table, idx_pages)


def _dotT(a, w):
    return lax.dot_general(a, w, (((1,), (1,)), ((), ())),
                           preferred_element_type=jnp.float32)


def _init_body(xv_ref, xc_ref, wv_ref, bv_ref, wc_ref, bc_ref,
               v_ref, c_ref):
    v_ref[...] = _dotT(xv_ref[...], wv_ref[...]) + bv_ref[...]
    c_ref[...] = _dotT(xc_ref[...], wc_ref[...]) + bc_ref[...]


def _init_linears(x, wv, bv, wc, bc):
    row = pl.BlockSpec((ROW_BLOCK, D), lambda i: (i, 0))
    crow = pl.BlockSpec((ROW_BLOCK, D), lambda i: (i + GRID, 0))
    full = pl.BlockSpec((D, D), lambda i: (0, 0))
    bias = pl.BlockSpec((1, D), lambda i: (0, 0))
    return pl.pallas_call(
        _init_body,
        grid=(GRID,),
        in_specs=[row, crow, full, bias, full, bias],
        out_specs=[row, row],
        out_shape=[jax.ShapeDtypeStruct((NV, D), jnp.float32),
                   jax.ShapeDtypeStruct((NC_NODES, D), jnp.float32)],
    )(x, x, wv, bv.reshape(1, D), wc, bc.reshape(1, D))


def _round_body(agg_ref, prev_ref, x_ref, w_ref, b_ref, y_ref):
    w = w_ref[...]
    y = _dotT(agg_ref[...], w[:, 0:D])
    y += _dotT(prev_ref[...], w[:, D:2 * D])
    y += _dotT(x_ref[...], w[:, 2 * D:3 * D])
    y_ref[...] = y + b_ref[...]


def _round_linear(agg, prev, x, x_off, w, b):
    row = pl.BlockSpec((ROW_BLOCK, D), lambda i: (i, 0))
    xrow = pl.BlockSpec((ROW_BLOCK, D), lambda i: (i + x_off, 0))
    wspec = pl.BlockSpec((D, 3 * D), lambda i: (0, 0))
    bias = pl.BlockSpec((1, D), lambda i: (0, 0))
    return pl.pallas_call(
        _round_body,
        grid=(GRID,),
        in_specs=[row, row, xrow, wspec, bias],
        out_specs=row,
        out_shape=jax.ShapeDtypeStruct((NV, D), jnp.float32),
    )(agg, prev, x, w, b.reshape(1, D))


def _final_v_body(agg_ref, prev_ref, xb_ref, w_ref, b_ref, w2_ref,
                  rowdot_ref, colsum_ref):
    w = w_ref[...]
    y = _dotT(agg_ref[...], w[:, 0:D])
    y += _dotT(prev_ref[...], w[:, D:2 * D])
    y += _dotT(xb_ref[...], w[:, 2 * D:3 * D])
    y = y + b_ref[...]
    rowdot_ref[...] = _dotT(y, w2_ref[...])
    i = pl.program_id(0)

    @pl.when(i == 0)
    def _():
        colsum_ref[...] = jnp.zeros_like(colsum_ref)

    colsum_ref[...] += jnp.sum(y, axis=0, keepdims=True)


def _final_v(agg, prev, xb, w, b, w2):
    row = pl.BlockSpec((ROW_BLOCK, D), lambda i: (i, 0))
    wspec = pl.BlockSpec((D, 3 * D), lambda i: (0, 0))
    bias = pl.BlockSpec((1, D), lambda i: (0, 0))
    return pl.pallas_call(
        _final_v_body,
        grid=(GRID,),
        in_specs=[row, row, row, wspec, bias, bias],
        out_specs=[pl.BlockSpec((ROW_BLOCK, 1), lambda i: (i, 0)),
                   pl.BlockSpec((1, D), lambda i: (0, 0))],
        out_shape=[jax.ShapeDtypeStruct((NV, 1), jnp.float32),
                   jax.ShapeDtypeStruct((1, D), jnp.float32)],
    )(agg, prev, xb, w, b.reshape(1, D), w2)


def _q_body(rowdot_ref, colsum_ref, w1_ref, bq_ref, xcol_ref, q_ref):
    s = jnp.sum(colsum_ref[...] * w1_ref[...]) + bq_ref[0, 0]
    q = rowdot_ref[...] + s
    mask = xcol_ref[...].astype(jnp.int32) != 0
    q_ref[...] = jnp.where(mask, jnp.inf, q)


def _q_stage(rowdot, colsum, w1, bq, xcol):
    row1 = pl.BlockSpec((ROW_BLOCK, 1), lambda i: (i, 0))
    return pl.pallas_call(
        _q_body,
        grid=(GRID,),
        in_specs=[row1,
                  pl.BlockSpec((1, D), lambda i: (0, 0)),
                  pl.BlockSpec((1, D), lambda i: (0, 0)),
                  pl.BlockSpec((1, 1), lambda i: (0, 0)),
                  row1],
        out_specs=row1,
        out_shape=jax.ShapeDtypeStruct((NV, 1), jnp.float32),
    )(rowdot, colsum, w1, bq.reshape(1, 1), xcol)


def kernel(x, var_constr_index, constr_var_index, W_init_var, b_init_var,
           W_init_con, b_init_con, W_var, b_var, W_con, b_con, W_q, b_q):
    xcol = x[:NV, 1:2]

    pad = PAD_N - NV
    vci = jnp.pad(var_constr_index, ((0, pad), (0, 0))).reshape(
        PAD_N * DEG // (BLK * DEG), BLK * DEG)
    cvi = jnp.pad(constr_var_index, ((0, pad), (0, 0))).reshape(
        PAD_N * DEG // (BLK * DEG), BLK * DEG)

    v32, c32 = _init_linears(x, W_init_var, b_init_var,
                             W_init_con, b_init_con)

    agg_c = _gather_sum(v32, cvi)[:NC_NODES]
    agg_v = _gather_sum(c32, vci)[:NV]
    nc32 = _round_linear(agg_c, c32, x, GRID, W_con, b_con)

    agg_v2 = _gather_sum(nc32, vci)[:NV]
    nv32 = _round_linear(agg_v, v32, x, 0, W_var, b_var)

    w1 = W_q[:, :D]
    w2 = W_q[:, D:]
    rowdot, colsum = _final_v(agg_v2, nv32, x[:NV], W_var, b_var, w2)
    return _q_stage(rowdot, colsum, w1, b_q, xcol)


# speedup vs baseline: 2.5986x; 2.5986x over previous
import functools

import jax
import jax.numpy as jnp
from jax import lax
from jax.experimental import pallas as pl
from jax.experimental.pallas import tpu as pltpu
from jax.experimental.pallas import tpu_sc as plsc

NV = 25000
NC_NODES = 25000
D = 128
DEG = 16
NW = 32
PAD_N = 25088
ROWS_PER_W = PAD_N // NW
BLK = 8
NBLK = ROWS_PER_W // BLK
ROW_BLOCK = 1000
GRID = NV // ROW_BLOCK


RING = 4
NGRP = (NBLK - 2) // RING


def _gather_pipeline(table_hbm, idx_v, out_hbm, base, bufs, gs, sos, out_v):
    for k in range(RING):
        pltpu.async_copy(table_hbm.at[idx_v.at[k]], bufs[k], gs[k])

    def compute_block(b, k, first_round):
        pltpu.make_async_copy(table_hbm.at[idx_v.at[b]], bufs[k],
                              gs[k]).wait()
        if first_round:
            drain = pl.when(b >= RING)
        else:
            drain = lambda f: f()

        @drain
        def _():
            pltpu.make_async_copy(out_v.at[pl.ds(k * BLK, BLK)],
                                  out_hbm.at[pl.ds(base, BLK)],
                                  sos[k]).wait()

        buf = bufs[k]

        def row_body(r, carry):
            for g in range(D // 16):
                vs = [buf[r * DEG + j, pl.ds(g * 16, 16)]
                      for j in range(DEG)]
                while len(vs) > 1:
                    vs = [vs[i] + vs[i + 1] for i in range(0, len(vs), 2)]
                out_v[k * BLK + r, pl.ds(g * 16, 16)] = vs[0]
            return carry

        lax.fori_loop(0, BLK, row_body, 0)
        pltpu.async_copy(out_v.at[pl.ds(k * BLK, BLK)],
                         out_hbm.at[pl.ds(base + b * BLK, BLK)], sos[k])

    def grp(q, carry):
        for k in range(RING):
            b = q * RING + k
            compute_block(b, k, first_round=True)

            @pl.when(b + RING < NBLK)
            def _():
                pltpu.async_copy(table_hbm.at[idx_v.at[b + RING]],
                                 bufs[k], gs[k])
        return carry

    lax.fori_loop(0, NGRP, grp, 0)
    compute_block(NBLK - 2, 0, first_round=False)
    compute_block(NBLK - 1, 1, first_round=False)
    for k in range(RING):
        pltpu.make_async_copy(out_v.at[pl.ds(k * BLK, BLK)],
                              out_hbm.at[pl.ds(base, BLK)], sos[k]).wait()


def _gather_sum_body(table_hbm, idx_hbm, out_hbm, idx_v,
                     b0, b1, b2, b3, out_v,
                     g0, g1, g2, g3, so0, so1, so2, so3):
    wid = lax.axis_index("s") * 2 + lax.axis_index("c")
    base = wid * ROWS_PER_W
    pltpu.sync_copy(idx_hbm.at[wid], idx_v)
    _gather_pipeline(table_hbm, idx_v, out_hbm, base,
                     [b0, b1, b2, b3], [g0, g1, g2, g3],
                     [so0, so1, so2, so3], out_v)


@functools.partial(jax.jit)
def _gather_sum(table, idx_pages):
    mesh = plsc.VectorSubcoreMesh(core_axis_name="c", subcore_axis_name="s")
    f = pl.kernel(
        _gather_sum_body,
        mesh=mesh,
        out_type=jax.ShapeDtypeStruct((PAD_N, D), jnp.float32),
        scratch_types=[
            pltpu.VMEM((NBLK, BLK * DEG), jnp.int32),
            pltpu.VMEM((BLK * DEG, D), jnp.float32),
            pltpu.VMEM((BLK * DEG, D), jnp.float32),
            pltpu.VMEM((BLK * DEG, D), jnp.float32),
            pltpu.VMEM((BLK * DEG, D), jnp.float32),
            pltpu.VMEM((RING * BLK, D), jnp.float32),
        ] + [pltpu.SemaphoreType.DMA] * 8,
    )
    return f(table, idx_pages)


def _dotT(a, w):
    return lax.dot_general(a, w, (((1,), (1,)), ((), ())),
                           preferred_element_type=jnp.float32)


def _init_body(xv_ref, xc_ref, wv_ref, bv_ref, wc_ref, bc_ref,
               v_ref, c_ref):
    v_ref[...] = _dotT(xv_ref[...], wv_ref[...]) + bv_ref[...]
    c_ref[...] = _dotT(xc_ref[...], wc_ref[...]) + bc_ref[...]


def _init_linears(x, wv, bv, wc, bc):
    row = pl.BlockSpec((ROW_BLOCK, D), lambda i: (i, 0))
    crow = pl.BlockSpec((ROW_BLOCK, D), lambda i: (i + GRID, 0))
    full = pl.BlockSpec((D, D), lambda i: (0, 0))
    bias = pl.BlockSpec((1, D), lambda i: (0, 0))
    return pl.pallas_call(
        _init_body,
        grid=(GRID,),
        in_specs=[row, crow, full, bias, full, bias],
        out_specs=[row, row],
        out_shape=[jax.ShapeDtypeStruct((NV, D), jnp.float32),
                   jax.ShapeDtypeStruct((NC_NODES, D), jnp.float32)],
    )(x, x, wv, bv.reshape(1, D), wc, bc.reshape(1, D))


def _round_body(agg_ref, prev_ref, x_ref, w_ref, b_ref, y_ref):
    w = w_ref[...]
    y = _dotT(agg_ref[...], w[:, 0:D])
    y += _dotT(prev_ref[...], w[:, D:2 * D])
    y += _dotT(x_ref[...], w[:, 2 * D:3 * D])
    y_ref[...] = y + b_ref[...]


def _round_linear(agg, prev, x, x_off, w, b):
    row = pl.BlockSpec((ROW_BLOCK, D), lambda i: (i, 0))
    xrow = pl.BlockSpec((ROW_BLOCK, D), lambda i: (i + x_off, 0))
    wspec = pl.BlockSpec((D, 3 * D), lambda i: (0, 0))
    bias = pl.BlockSpec((1, D), lambda i: (0, 0))
    return pl.pallas_call(
        _round_body,
        grid=(GRID,),
        in_specs=[row, row, xrow, wspec, bias],
        out_specs=row,
        out_shape=jax.ShapeDtypeStruct((NV, D), jnp.float32),
    )(agg, prev, x, w, b.reshape(1, D))


def _final_v_body(agg_ref, prev_ref, xb_ref, w_ref, b_ref, w2_ref,
                  rowdot_ref, colsum_ref):
    w = w_ref[...]
    y = _dotT(agg_ref[...], w[:, 0:D])
    y += _dotT(prev_ref[...], w[:, D:2 * D])
    y += _dotT(xb_ref[...], w[:, 2 * D:3 * D])
    y = y + b_ref[...]
    rowdot_ref[...] = _dotT(y, w2_ref[...])
    i = pl.program_id(0)

    @pl.when(i == 0)
    def _():
        colsum_ref[...] = jnp.zeros_like(colsum_ref)

    colsum_ref[...] += jnp.sum(y, axis=0, keepdims=True)


def _final_v(agg, prev, xb, w, b, w2):
    row = pl.BlockSpec((ROW_BLOCK, D), lambda i: (i, 0))
    wspec = pl.BlockSpec((D, 3 * D), lambda i: (0, 0))
    bias = pl.BlockSpec((1, D), lambda i: (0, 0))
    return pl.pallas_call(
        _final_v_body,
        grid=(GRID,),
        in_specs=[row, row, row, wspec, bias, bias],
        out_specs=[pl.BlockSpec((ROW_BLOCK, 1), lambda i: (i, 0)),
                   pl.BlockSpec((1, D), lambda i: (0, 0))],
        out_shape=[jax.ShapeDtypeStruct((NV, 1), jnp.float32),
                   jax.ShapeDtypeStruct((1, D), jnp.float32)],
    )(agg, prev, xb, w, b.reshape(1, D), w2)


def _q_body(rowdot_ref, colsum_ref, w1_ref, bq_ref, xcol_ref, q_ref):
    s = jnp.sum(colsum_ref[...] * w1_ref[...]) + bq_ref[0, 0]
    q = rowdot_ref[...] + s
    mask = xcol_ref[...].astype(jnp.int32) != 0
    q_ref[...] = jnp.where(mask, jnp.inf, q)


def _q_stage(rowdot, colsum, w1, bq, xcol):
    row1 = pl.BlockSpec((ROW_BLOCK, 1), lambda i: (i, 0))
    return pl.pallas_call(
        _q_body,
        grid=(GRID,),
        in_specs=[row1,
                  pl.BlockSpec((1, D), lambda i: (0, 0)),
                  pl.BlockSpec((1, D), lambda i: (0, 0)),
                  pl.BlockSpec((1, 1), lambda i: (0, 0)),
                  row1],
        out_specs=row1,
        out_shape=jax.ShapeDtypeStruct((NV, 1), jnp.float32),
    )(rowdot, colsum, w1, bq.reshape(1, 1), xcol)


def kernel(x, var_constr_index, constr_var_index, W_init_var, b_init_var,
           W_init_con, b_init_con, W_var, b_var, W_con, b_con, W_q, b_q):
    xcol = x[:NV, 1:2]

    pad = PAD_N - NV
    vci = jnp.pad(var_constr_index, ((0, pad), (0, 0))).reshape(
        NW, NBLK, BLK * DEG)
    cvi = jnp.pad(constr_var_index, ((0, pad), (0, 0))).reshape(
        NW, NBLK, BLK * DEG)

    v32, c32 = _init_linears(x, W_init_var, b_init_var,
                             W_init_con, b_init_con)

    agg_c = _gather_sum(v32, cvi)[:NC_NODES]
    agg_v = _gather_sum(c32, vci)[:NV]
    nc32 = _round_linear(agg_c, c32, x, GRID, W_con, b_con)

    agg_v2 = _gather_sum(nc32, vci)[:NV]
    nv32 = _round_linear(agg_v, v32, x, 0, W_var, b_var)

    w1 = W_q[:, :D]
    w2 = W_q[:, D:]
    rowdot, colsum = _final_v(agg_v2, nv32, x[:NV], W_var, b_var, w2)
    return _q_stage(rowdot, colsum, w1, b_q, xcol)
